# baseline (speedup 1.0000x reference)
import jax
import jax.numpy as jnp
from jax.experimental import pallas as pl

SEQ_BLOCK = 512


def _add_kernel(x_ref, pos_ref, o_ref):
    p = pos_ref[...]
    o_ref[...] = x_ref[...] + jnp.concatenate([p, p, p, p], axis=1)


def kernel(x, pos_embed_weight):
    S, B, D = x.shape
    pos = pos_embed_weight[:S]
    x2 = x.reshape(S, B * D)
    out = pl.pallas_call(
        _add_kernel,
        grid=(S // SEQ_BLOCK,),
        in_specs=[
            pl.BlockSpec((SEQ_BLOCK, B * D), lambda i: (i, 0)),
            pl.BlockSpec((SEQ_BLOCK, D), lambda i: (i, 0)),
        ],
        out_specs=pl.BlockSpec((SEQ_BLOCK, B * D), lambda i: (i, 0)),
        out_shape=jax.ShapeDtypeStruct((S, B * D), x.dtype),
    )(x2, pos)
    return out.reshape(S, B, D)


# speedup vs baseline: 3.7865x; 3.7865x over previous
import jax
import jax.numpy as jnp
from jax.experimental import pallas as pl

SEQ_BLOCK = 256


def _add_kernel(x_ref, pos_ref, o_ref):
    pos = pos_ref[...]
    o_ref[...] = x_ref[...] + pos[:, None, :]


def kernel(x, pos_embed_weight):
    S, B, D = x.shape
    pos = pos_embed_weight[:S]
    return pl.pallas_call(
        _add_kernel,
        grid=(S // SEQ_BLOCK,),
        in_specs=[
            pl.BlockSpec((SEQ_BLOCK, B, D), lambda i: (i, 0, 0)),
            pl.BlockSpec((SEQ_BLOCK, D), lambda i: (i, 0)),
        ],
        out_specs=pl.BlockSpec((SEQ_BLOCK, B, D), lambda i: (i, 0, 0)),
        out_shape=jax.ShapeDtypeStruct((S, B, D), x.dtype),
    )(x, pos)


# speedup vs baseline: 3.8294x; 1.0113x over previous
import jax
import jax.numpy as jnp
from jax.experimental import pallas as pl

SEQ_BLOCK = 512


def _add_kernel(x_ref, pos_ref, o_ref):
    pos = pos_ref[...]
    o_ref[...] = x_ref[...] + pos[:, None, :]


def kernel(x, pos_embed_weight):
    S, B, D = x.shape
    pos = pos_embed_weight[:S]
    return pl.pallas_call(
        _add_kernel,
        grid=(S // SEQ_BLOCK,),
        in_specs=[
            pl.BlockSpec((SEQ_BLOCK, B, D), lambda i: (i, 0, 0)),
            pl.BlockSpec((SEQ_BLOCK, D), lambda i: (i, 0)),
        ],
        out_specs=pl.BlockSpec((SEQ_BLOCK, B, D), lambda i: (i, 0, 0)),
        out_shape=jax.ShapeDtypeStruct((S, B, D), x.dtype),
    )(x, pos)
